# baseline (device time: 972256 ns/iter reference)
import jax
import jax.numpy as jnp
from jax import lax
from jax.experimental import pallas as pl
from jax.experimental.pallas import tpu as pltpu

N_DEV = 32
E_LOCAL = 4
N_EXPERTS = 128
N_TOK = 1024
D_MODEL = 256
D_HID = 512


def kernel(x, router_W, route_idx, expert_W, shared_W):
    def body(x_ref, router_ref, idx_ref, ew_ref, sw_ref, out_ref,
             comm_ref, send_sems, recv_sems, credit_sem):
        my = lax.axis_index("i")
        left = lax.rem(my - 1 + N_DEV, N_DEV)
        right = lax.rem(my + 1, N_DEV)

        barrier_sem = pltpu.get_barrier_semaphore()
        for nbr in (left, right):
            pl.semaphore_signal(
                barrier_sem, inc=1,
                device_id=(nbr,), device_id_type=pl.DeviceIdType.MESH,
            )
        pl.semaphore_wait(barrier_sem, 2)

        xv = x_ref[:, :]
        scores = jnp.dot(xv, router_ref[:, :],
                         preferred_element_type=jnp.float32)
        m = jnp.max(scores, axis=-1, keepdims=True)
        e = jnp.exp(scores - m)
        probs = e / jnp.sum(e, axis=-1, keepdims=True)
        idx = idx_ref[:, 0]
        eids = lax.broadcasted_iota(jnp.int32, (N_TOK, N_EXPERTS), 1)
        p_tok = jnp.sum(jnp.where(eids == idx[:, None], probs, 0.0),
                        axis=-1)

        e4 = lax.broadcasted_iota(jnp.int32, (N_TOK, E_LOCAL), 1)

        def contribution(src, W):
            mask = idx[:, None] == (e4 + src * E_LOCAL)
            sl = jnp.where(mask, p_tok[:, None], 0.0)
            acc = jnp.zeros((N_TOK, D_HID), jnp.float32)
            for ei in range(E_LOCAL):
                xm = xv * sl[:, ei][:, None]
                acc = acc + jnp.dot(xm, W[ei],
                                    preferred_element_type=jnp.float32)
            return acc

        out_ref[:, :] = (
            jnp.dot(xv, sw_ref[:, :], preferred_element_type=jnp.float32)
            + contribution(my, ew_ref[:, :, :])
        )

        comm_ref[0, :, :, :] = ew_ref[:, :, :]

        for h in range(N_DEV - 1):
            send_slot = h % 2
            recv_slot = (h + 1) % 2
            if h >= 1:
                pl.semaphore_wait(credit_sem, 1)
            rdma = pltpu.make_async_remote_copy(
                src_ref=comm_ref.at[send_slot],
                dst_ref=comm_ref.at[recv_slot],
                send_sem=send_sems.at[send_slot],
                recv_sem=recv_sems.at[recv_slot],
                device_id=(right,),
                device_id_type=pl.DeviceIdType.MESH,
            )
            rdma.start()
            rdma.wait()

            src = lax.rem(my - (h + 1) + N_DEV, N_DEV)
            out_ref[:, :] = out_ref[:, :] + contribution(
                src, comm_ref[recv_slot, :, :, :])

            pl.semaphore_signal(
                credit_sem, inc=1,
                device_id=(left,), device_id_type=pl.DeviceIdType.MESH,
            )

        pl.semaphore_wait(credit_sem, 1)

    return pl.pallas_call(
        body,
        out_shape=jax.ShapeDtypeStruct((N_TOK, D_HID), jnp.float32),
        in_specs=[
            pl.BlockSpec(memory_space=pltpu.VMEM),
            pl.BlockSpec(memory_space=pltpu.VMEM),
            pl.BlockSpec(memory_space=pltpu.VMEM),
            pl.BlockSpec(memory_space=pltpu.VMEM),
            pl.BlockSpec(memory_space=pltpu.VMEM),
        ],
        out_specs=pl.BlockSpec(memory_space=pltpu.VMEM),
        scratch_shapes=[
            pltpu.VMEM((2, E_LOCAL, D_MODEL, D_HID), jnp.float32),
            pltpu.SemaphoreType.DMA((2,)),
            pltpu.SemaphoreType.DMA((2,)),
            pltpu.SemaphoreType.REGULAR,
        ],
        compiler_params=pltpu.CompilerParams(collective_id=0),
    )(x, router_W, route_idx, expert_W, shared_W)


# device time: 776471 ns/iter; 1.2521x vs baseline; 1.2521x over previous
import jax
import jax.numpy as jnp
from jax import lax
from jax.experimental import pallas as pl
from jax.experimental.pallas import tpu as pltpu

N_DEV = 32
E_LOCAL = 4
N_EXPERTS = 128
N_TOK = 1024
D_MODEL = 256
D_HID = 512

N_R = 16
N_L = 15


def kernel(x, router_W, route_idx, expert_W, shared_W):
    def body(x_ref, router_ref, idx_ref, ew_ref, sw_ref, out_ref,
             commR, commL, sendR_sems, recvR_sems, sendL_sems, recvL_sems,
             creditR, creditL):
        my = lax.axis_index("i")
        left = lax.rem(my - 1 + N_DEV, N_DEV)
        right = lax.rem(my + 1, N_DEV)

        barrier_sem = pltpu.get_barrier_semaphore()
        for nbr in (left, right):
            pl.semaphore_signal(
                barrier_sem, inc=1,
                device_id=(nbr,), device_id_type=pl.DeviceIdType.MESH,
            )
        pl.semaphore_wait(barrier_sem, 2)

        def descR(h):
            return pltpu.make_async_remote_copy(
                src_ref=commR.at[h % 2],
                dst_ref=commR.at[(h + 1) % 2],
                send_sem=sendR_sems.at[h % 2],
                recv_sem=recvR_sems.at[(h + 1) % 2],
                device_id=(right,),
                device_id_type=pl.DeviceIdType.MESH,
            )

        def descL(h):
            return pltpu.make_async_remote_copy(
                src_ref=commL.at[h % 2],
                dst_ref=commL.at[(h + 1) % 2],
                send_sem=sendL_sems.at[h % 2],
                recv_sem=recvL_sems.at[(h + 1) % 2],
                device_id=(left,),
                device_id_type=pl.DeviceIdType.MESH,
            )

        commR[0, :, :, :] = ew_ref[:, :, :]
        commL[0, :, :, :] = ew_ref[:, :, :]
        descR(0).start()
        descL(0).start()

        xv = x_ref[:, :]
        scores = jnp.dot(xv, router_ref[:, :],
                         preferred_element_type=jnp.float32)
        m = jnp.max(scores, axis=-1, keepdims=True)
        e = jnp.exp(scores - m)
        probs = e / jnp.sum(e, axis=-1, keepdims=True)
        idx = idx_ref[:, 0]
        eids = lax.broadcasted_iota(jnp.int32, (N_TOK, N_EXPERTS), 1)
        p_tok = jnp.sum(jnp.where(eids == idx[:, None], probs, 0.0),
                        axis=-1)

        e4 = lax.broadcasted_iota(jnp.int32, (N_TOK, E_LOCAL), 1)

        def contribution(src, W):
            mask = idx[:, None] == (e4 + src * E_LOCAL)
            sl = jnp.where(mask, p_tok[:, None], 0.0)
            acc = jnp.zeros((N_TOK, D_HID), jnp.float32)
            for ei in range(E_LOCAL):
                xm = xv * sl[:, ei][:, None]
                acc = acc + jnp.dot(xm, W[ei],
                                    preferred_element_type=jnp.float32)
            return acc

        out_ref[:, :] = (
            jnp.dot(xv, sw_ref[:, :], preferred_element_type=jnp.float32)
            + contribution(my, ew_ref[:, :, :])
        )

        for h in range(N_R):
            descR(h).wait_send()
            if h < N_L:
                descL(h).wait_send()
            pl.semaphore_signal(
                creditR, inc=1,
                device_id=(left,), device_id_type=pl.DeviceIdType.MESH,
            )
            if h < N_L:
                pl.semaphore_signal(
                    creditL, inc=1,
                    device_id=(right,), device_id_type=pl.DeviceIdType.MESH,
                )

            descR(h).wait_recv()
            if h + 1 < N_R:
                pl.semaphore_wait(creditR, 1)
                descR(h + 1).start()
            if h < N_L:
                descL(h).wait_recv()
            if h + 1 < N_L:
                pl.semaphore_wait(creditL, 1)
                descL(h + 1).start()

            srcR = lax.rem(my - (h + 1) + N_DEV, N_DEV)
            out_ref[:, :] = out_ref[:, :] + contribution(
                srcR, commR[(h + 1) % 2, :, :, :])
            if h < N_L:
                srcL = lax.rem(my + (h + 1), N_DEV)
                out_ref[:, :] = out_ref[:, :] + contribution(
                    srcL, commL[(h + 1) % 2, :, :, :])

        pl.semaphore_wait(creditR, 1)
        pl.semaphore_wait(creditL, 1)

    return pl.pallas_call(
        body,
        out_shape=jax.ShapeDtypeStruct((N_TOK, D_HID), jnp.float32),
        in_specs=[
            pl.BlockSpec(memory_space=pltpu.VMEM),
            pl.BlockSpec(memory_space=pltpu.VMEM),
            pl.BlockSpec(memory_space=pltpu.VMEM),
            pl.BlockSpec(memory_space=pltpu.VMEM),
            pl.BlockSpec(memory_space=pltpu.VMEM),
        ],
        out_specs=pl.BlockSpec(memory_space=pltpu.VMEM),
        scratch_shapes=[
            pltpu.VMEM((2, E_LOCAL, D_MODEL, D_HID), jnp.float32),
            pltpu.VMEM((2, E_LOCAL, D_MODEL, D_HID), jnp.float32),
            pltpu.SemaphoreType.DMA((2,)),
            pltpu.SemaphoreType.DMA((2,)),
            pltpu.SemaphoreType.DMA((2,)),
            pltpu.SemaphoreType.DMA((2,)),
            pltpu.SemaphoreType.REGULAR,
            pltpu.SemaphoreType.REGULAR,
        ],
        compiler_params=pltpu.CompilerParams(collective_id=0),
    )(x, router_W, route_idx, expert_W, shared_W)
